# baseline (speedup 1.0000x reference)
import functools

import jax
import jax.numpy as jnp
from jax import lax
from jax.experimental import pallas as pl
from jax.experimental.pallas import tpu as pltpu
from jax.experimental.pallas import tpu_sc as plsc

N_FACTORS = 64
BATCH = 16384
N_ROWS = 1000000
NUM_CORES = 2
NUM_SUBCORES = 16
NW = NUM_CORES * NUM_SUBCORES
BPW = BATCH // NW
LANES = 16
BLK = 128
NBF = N_ROWS // BLK
TAIL_LO = NBF * BLK
NTAIL = N_ROWS - TAIL_LO
BASE_NB = NBF // NW
EXTRA = NBF - BASE_NB * NW
MAXNB = BASE_NB + 1
IDX_CH = 2048
CHUNK = 128
CH = 256

_mesh = plsc.VectorSubcoreMesh(
    core_axis_name="c", subcore_axis_name="s",
    num_cores=NUM_CORES, num_subcores=NUM_SUBCORES)

_i32 = jnp.int32


def _splat(x):
    return jnp.zeros((LANES,), _i32) + x


@functools.partial(
    pl.kernel,
    out_type=[jax.ShapeDtypeStruct((BATCH, N_FACTORS), jnp.float32),
              jax.ShapeDtypeStruct((BATCH, N_FACTORS), jnp.float32)],
    mesh=_mesh,
    compiler_params=pltpu.CompilerParams(needs_layout_passes=False),
    scratch_types=[
        pltpu.VMEM((IDX_CH,), _i32),
        pltpu.VMEM((BATCH + LANES,), _i32),
        pltpu.VMEM((BATCH + LANES,), _i32),
        pltpu.VMEM((BATCH + LANES,), _i32),
        pltpu.VMEM((BATCH + LANES,), _i32),
        pltpu.VMEM((N_FACTORS, BLK), jnp.float32),
        pltpu.VMEM((N_FACTORS, BLK), jnp.float32),
        pltpu.VMEM((N_FACTORS, BLK), jnp.float32),
        pltpu.VMEM((N_FACTORS, BLK), jnp.float32),
        pltpu.VMEM((NTAIL, N_FACTORS), jnp.float32),
        pltpu.VMEM((NTAIL, N_FACTORS), jnp.float32),
        pltpu.VMEM((LANES,), _i32),
        pltpu.VMEM((LANES,), _i32),
        pltpu.VMEM((LANES, N_FACTORS), jnp.float32),
        pltpu.VMEM((LANES, N_FACTORS), jnp.float32),
        pltpu.SMEM((4,), _i32),
        pltpu.SemaphoreType.DMA,
        pltpu.SemaphoreType.DMA,
        pltpu.SemaphoreType.DMA,
        pltpu.SemaphoreType.DMA,
    ],
)
def _stage(u_idx_hbm, i_idx_hbm, ue_t, ie_t, tail_u, tail_i,
           ug_hbm, ig_hbm, idx_ch, mu_r, mu_p, mi_r, mi_p,
           slab_u0, slab_u1, slab_i0, slab_i1, tail_u_v, tail_i_v,
           t_r, t_p, stage0, stage1, ctrs, sem_sl0, sem_sl1, sem_o0,
           sem_o1):
    w = lax.axis_index("s") * NUM_CORES + lax.axis_index("c")
    b_base = w * BASE_NB + jnp.minimum(w, EXTRA)
    nb = BASE_NB + jnp.where(w < EXTRA, 1, 0)
    lo = b_base * BLK
    hi_scan = jnp.where(w == NW - 1, N_ROWS, (b_base + nb) * BLK)

    lane = lax.iota(_i32, LANES)
    slabs = ((slab_u0, slab_i0, sem_sl0), (slab_u1, slab_i1, sem_sl1))
    stages = ((stage0, sem_o0), (stage1, sem_o1))

    pltpu.sync_copy(tail_u, tail_u_v)
    pltpu.sync_copy(tail_i, tail_i_v)

    def scan_list(idx_hbm, m_r, m_p):
        def cc_body(cc, off):
            pltpu.sync_copy(idx_hbm.at[pl.ds(cc * IDX_CH, IDX_CH)], idx_ch)

            def k_body(k, off2):
                rv = idx_ch[pl.ds(k * LANES, LANES)]
                posv = cc * IDX_CH + k * LANES + lane
                m = (rv >= lo) & (rv < hi_scan)
                plsc.store_compressed(m_r.at[pl.ds(off2, LANES)], rv, mask=m)
                plsc.store_compressed(m_p.at[pl.ds(off2, LANES)], posv, mask=m)
                cnt = plsc.all_reduce_population_count(m)
                return off2 + cnt[0]

            return lax.fori_loop(0, IDX_CH // LANES, k_body, off)

        return lax.fori_loop(0, BATCH // IDX_CH, cc_body, 0)

    cnt_u = scan_list(u_idx_hbm, mu_r, mu_p)
    cnt_i = scan_list(i_idx_hbm, mi_r, mi_p)

    def fire_slabs(k, par):
        off = (b_base + k) * BLK
        su, si, sem = slabs[par]
        pltpu.make_async_copy(
            ue_t.at[:, pl.ds(off, BLK)], su, sem).start()
        pltpu.make_async_copy(
            ie_t.at[:, pl.ds(off, BLK)], si, sem).start()

    def wait_slabs(par):
        su, si, sem = slabs[par]
        pltpu.make_async_copy(ue_t.at[:, pl.ds(0, BLK)], su, sem).wait()
        pltpu.make_async_copy(ie_t.at[:, pl.ds(0, BLK)], si, sem).wait()

    def drain_out(par, n):
        stg, sem = stages[par]

        def d_body(d, c):
            pltpu.make_async_copy(
                stg.at[pl.ds(0, 1), :], ug_hbm.at[pl.ds(0, 1), :],
                sem).wait()
            return c

        lax.fori_loop(0, n, d_body, 0)

    ctrs[0] = 0
    ctrs[1] = 0
    ctrs[2] = 0

    f_lanes = [lane + f0 * LANES for f0 in range(4)]

    def extract_matches(m_r, m_p, cnt, blo, nrows, reader, g_hbm):
        ngrp = (cnt + LANES - 1) // LANES

        def g_body(g, car):
            mr = m_r[pl.ds(g * LANES, LANES)]
            mp = m_p[pl.ds(g * LANES, LANES)]
            valid = (g * LANES + lane) < cnt
            inb = (mr >= blo) & (mr < blo + nrows) & valid
            hv = plsc.all_reduce_population_count(inb)[0]

            def do_group(par):
                stg, sem = stages[par]
                drain_out(par, ctrs[1 + par])
                plsc.store_compressed(t_r.at[pl.ds(0, LANES)], mr, mask=inb)
                plsc.store_compressed(t_p.at[pl.ds(0, LANES)], mp, mask=inb)

                def m_body(m, c):
                    rr = plsc.load_gather(t_r, [_splat(m)])[0]
                    pp = plsc.load_gather(t_p, [_splat(m)])[0]
                    vecs = reader(rr - blo)
                    for f0 in range(4):
                        plsc.store_scatter(stg, [_splat(m), f_lanes[f0]],
                                           vecs[f0])
                    pltpu.make_async_copy(
                        stg.at[pl.ds(m, 1), :],
                        g_hbm.at[pl.ds(pp, 1), :], sem).start()
                    return c

                lax.fori_loop(0, hv, m_body, 0)
                ctrs[1 + par] = hv
                ctrs[0] = ctrs[0] + 1

            @pl.when((hv > 0) & (ctrs[0] % 2 == 0))
            def _():
                do_group(0)

            @pl.when((hv > 0) & (ctrs[0] % 2 == 1))
            def _():
                do_group(1)

            return car

        lax.fori_loop(0, ngrp, g_body, 0)

    def slab_reader(slab):
        def read(lr):
            col = _splat(lr)
            return [plsc.load_gather(slab, [f_lanes[f0], col])
                    for f0 in range(4)]
        return read

    def tail_reader(tv):
        def read(lr):
            rowv = _splat(lr)
            return [plsc.load_gather(tv, [rowv, f_lanes[f0]])
                    for f0 in range(4)]
        return read

    def process_block(k, par):
        wait_slabs(par)

        @pl.when(k + 1 < nb)
        def _():
            fire_slabs(k + 1, 1 - par)

        blo = (b_base + k) * BLK
        su, si, _ = slabs[par]
        extract_matches(mu_r, mu_p, cnt_u, blo, BLK, slab_reader(su),
                        ug_hbm)
        extract_matches(mi_r, mi_p, cnt_i, blo, BLK, slab_reader(si),
                        ig_hbm)

    fire_slabs(0, 0)

    def blk_body(k2, car):
        process_block(k2 * 2, 0)
        process_block(k2 * 2 + 1, 1)
        return car

    lax.fori_loop(0, BASE_NB // 2, blk_body, 0)

    @pl.when(nb > BASE_NB)
    def _():
        process_block(BASE_NB, 0)

    extract_matches(mu_r, mu_p, cnt_u, TAIL_LO, NTAIL,
                    tail_reader(tail_u_v), ug_hbm)
    extract_matches(mi_r, mi_p, cnt_i, TAIL_LO, NTAIL,
                    tail_reader(tail_i_v), ig_hbm)

    drain_out(0, ctrs[1])
    drain_out(1, ctrs[2])


@functools.partial(
    pl.kernel,
    out_type=jax.ShapeDtypeStruct((BATCH,), jnp.float32),
    mesh=_mesh,
    compiler_params=pltpu.CompilerParams(needs_layout_passes=False),
    scratch_types=[
        pltpu.VMEM((BPW,), _i32),
        pltpu.VMEM((BPW,), _i32),
        pltpu.VMEM((CH, N_FACTORS), jnp.float32),
        pltpu.VMEM((CH, N_FACTORS), jnp.float32),
        pltpu.VMEM((BPW,), jnp.float32),
        pltpu.VMEM((BPW,), jnp.float32),
        pltpu.VMEM((BPW,), jnp.float32),
        pltpu.SemaphoreType.DMA,
        pltpu.SemaphoreType.DMA,
    ],
)
def _dot(u_idx_hbm, i_idx_hbm, ug_hbm, ig_hbm, ub_hbm, ib_hbm,
         out_hbm, uidx_v, iidx_v, ue_v, ie_v, ub_v, ib_v,
         out_v, sem_rows, sem_bias):
    wid = lax.axis_index("s") * NUM_CORES + lax.axis_index("c")
    base = wid * BPW

    pltpu.sync_copy(u_idx_hbm.at[pl.ds(base, BPW)], uidx_v)
    pltpu.sync_copy(i_idx_hbm.at[pl.ds(base, BPW)], iidx_v)

    bias_copies = []
    for c in range(BPW // CHUNK):
        sl = pl.ds(c * CHUNK, CHUNK)
        bias_copies.append(pltpu.async_copy(
            ub_hbm.at[0].at[uidx_v.at[sl]], ub_v.at[sl], sem_bias))
        bias_copies.append(pltpu.async_copy(
            ib_hbm.at[0].at[iidx_v.at[sl]], ib_v.at[sl], sem_bias))

    lane = lax.iota(_i32, LANES)

    def run_pass(p, carry):
        p0 = p * CH
        cu = pltpu.async_copy(
            ug_hbm.at[pl.ds(base + p0, CH), :], ue_v, sem_rows)
        ci = pltpu.async_copy(
            ig_hbm.at[pl.ds(base + p0, CH), :], ie_v, sem_rows)
        cu.wait()
        ci.wait()

        def group_body(g, carry2):
            r0 = g * LANES
            rows = r0 + lane
            o0 = p0 + r0
            acc = ub_v[pl.ds(o0, LANES)] + ib_v[pl.ds(o0, LANES)]
            for f in range(N_FACTORS):
                fv = jnp.full((LANES,), f, _i32)
                a = plsc.load_gather(ue_v, [rows, fv])
                b = plsc.load_gather(ie_v, [rows, fv])
                acc = acc + a * b
            out_v[pl.ds(o0, LANES)] = acc
            return carry2

        lax.fori_loop(0, CH // LANES, group_body, 0)
        return carry

    for cp in bias_copies:
        cp.wait()
    lax.fori_loop(0, BPW // CH, run_pass, 0)

    pltpu.sync_copy(out_v, out_hbm.at[pl.ds(base, BPW)])


def kernel(u_idx, i_idx, u_emb, i_emb, u_bias, i_bias):
    ue_g, ie_g = _stage(u_idx, i_idx, u_emb.T, i_emb.T,
                        u_emb[TAIL_LO:], i_emb[TAIL_LO:])
    return _dot(u_idx, i_idx, ue_g, ie_g, u_bias.T, i_bias.T)


# speedup vs baseline: 1.0923x; 1.0923x over previous
import functools

import jax
import jax.numpy as jnp
from jax import lax
from jax.experimental import pallas as pl
from jax.experimental.pallas import tpu as pltpu
from jax.experimental.pallas import tpu_sc as plsc

N_FACTORS = 64
BATCH = 16384
NUM_CORES = 2
NUM_SUBCORES = 16
NW = NUM_CORES * NUM_SUBCORES
BPW = BATCH // NW
CH = 256
CHUNK = 128
LANES = 16
ROW_UNROLL = 16

_mesh = plsc.VectorSubcoreMesh(
    core_axis_name="c", subcore_axis_name="s",
    num_cores=NUM_CORES, num_subcores=NUM_SUBCORES)


@functools.partial(
    pl.kernel,
    out_type=jax.ShapeDtypeStruct((BATCH,), jnp.float32),
    mesh=_mesh,
    compiler_params=pltpu.CompilerParams(needs_layout_passes=False),
    scratch_types=[
        pltpu.VMEM((BPW,), jnp.int32),
        pltpu.VMEM((BPW,), jnp.int32),
        pltpu.VMEM((CH, N_FACTORS), jnp.float32),
        pltpu.VMEM((CH, N_FACTORS), jnp.float32),
        pltpu.VMEM((BPW,), jnp.float32),
        pltpu.VMEM((BPW,), jnp.float32),
        pltpu.VMEM((BPW,), jnp.float32),
        pltpu.SemaphoreType.DMA,
        pltpu.SemaphoreType.DMA,
    ],
)
def _mf_sc(u_idx_hbm, i_idx_hbm, u_emb_hbm, i_emb_hbm, ub_hbm, ib_hbm,
           out_hbm, uidx_v, iidx_v, ue_v, ie_v, ub_v, ib_v,
           out_v, sem_rows, sem_bias):
    wid = lax.axis_index("s") * NUM_CORES + lax.axis_index("c")
    base = wid * BPW

    pltpu.sync_copy(u_idx_hbm.at[pl.ds(base, BPW)], uidx_v)
    pltpu.sync_copy(i_idx_hbm.at[pl.ds(base, BPW)], iidx_v)

    bias_copies = []
    for c in range(BPW // CHUNK):
        sl = pl.ds(c * CHUNK, CHUNK)
        bias_copies.append(pltpu.async_copy(
            ub_hbm.at[0].at[uidx_v.at[sl]], ub_v.at[sl], sem_bias))
        bias_copies.append(pltpu.async_copy(
            ib_hbm.at[0].at[iidx_v.at[sl]], ib_v.at[sl], sem_bias))

    lane = lax.iota(jnp.int32, LANES)

    def run_pass(p, carry):
        p0 = p * CH

        def fire_chunk(c, carry2):
            i0 = c * ROW_UNROLL
            uvec = uidx_v[pl.ds(p0 + i0, ROW_UNROLL)]
            ivec = iidx_v[pl.ds(p0 + i0, ROW_UNROLL)]
            for j in range(ROW_UNROLL):
                i = i0 + j
                ru = uvec[j]
                ri = ivec[j]
                pltpu.make_async_copy(
                    u_emb_hbm.at[pl.ds(ru, 1), :], ue_v.at[pl.ds(i, 1), :],
                    sem_rows).start()
                pltpu.make_async_copy(
                    i_emb_hbm.at[pl.ds(ri, 1), :], ie_v.at[pl.ds(i, 1), :],
                    sem_rows).start()
            return carry2

        lax.fori_loop(0, CH // ROW_UNROLL, fire_chunk, 0)
        pltpu.make_async_copy(
            u_emb_hbm.at[pl.ds(0, CH), :], ue_v, sem_rows).wait()
        pltpu.make_async_copy(
            i_emb_hbm.at[pl.ds(0, CH), :], ie_v, sem_rows).wait()

        def group_body(g, carry2):
            r0 = g * LANES
            rows = r0 + lane
            o0 = p0 + r0
            acc = ub_v[pl.ds(o0, LANES)] + ib_v[pl.ds(o0, LANES)]
            for f in range(N_FACTORS):
                fv = jnp.full((LANES,), f, jnp.int32)
                a = plsc.load_gather(ue_v, [rows, fv])
                b = plsc.load_gather(ie_v, [rows, fv])
                acc = acc + a * b
            out_v[pl.ds(o0, LANES)] = acc
            return carry2

        lax.fori_loop(0, CH // LANES, group_body, 0)
        return carry

    for cp in bias_copies:
        cp.wait()
    lax.fori_loop(0, BPW // CH, run_pass, 0)

    pltpu.sync_copy(out_v, out_hbm.at[pl.ds(base, BPW)])


def kernel(u_idx, i_idx, u_emb, i_emb, u_bias, i_bias):
    return _mf_sc(u_idx, i_idx, u_emb, i_emb, u_bias.T, i_bias.T)
